# baseline (speedup 1.0000x reference)
import functools

import jax
import jax.numpy as jnp
from jax import lax
from jax.experimental import pallas as pl
from jax.experimental.pallas import tpu as pltpu
from jax.experimental.pallas import tpu_sc as plsc

_B = 16
_M = 1024
_K = 16
_F = 256
_N = 4096

_H = 8
_FH = _F // 2
_NC = 2
_NS = 16
_NW = _NC * _NS
_RPW = (_H * _M) // _NW
_G = 8
_NCH = _RPW // _G
_IPC = _G * _K
_LANES = 16
_NBUF = 4

_HIMASK = -65536


def _body(
    tbl, idx, out, idxv, rows, outv,
    gsem0, gsem1, gsem2, gsem3, osem0, osem1, osem2, osem3,
):
    wid = lax.axis_index("s") * _NC + lax.axis_index("c")
    boff = (wid // (_NW // _H)) * _N

    pltpu.sync_copy(idx.at[wid], idxv)

    def add_off(i, _):
        for j in range(_IPC // _LANES):
            sl = (i, pl.ds(j * _LANES, _LANES))
            idxv[sl] = idxv[sl] + boff
        return 0

    lax.fori_loop(0, _NCH, add_off, 0)

    gsems = (gsem0, gsem1, gsem2, gsem3)
    osems = (osem0, osem1, osem2, osem3)

    def start_gather(c, buf):
        pltpu.async_copy(tbl.at[idxv.at[c]], rows.at[buf], gsems[buf])

    def wait_gather(c, buf):
        pltpu.make_async_copy(tbl.at[idxv.at[c]], rows.at[buf], gsems[buf]).wait()

    def wait_out(buf):
        pltpu.make_async_copy(outv.at[buf], out.at[pl.ds(0, _G)], osems[buf]).wait()

    def compute(c, buf):
        @pl.when(c >= _NBUF)
        def _():
            wait_out(buf)

        def per_g(g, _):
            r0 = g * _K

            def per_d(d, __):
                s = pl.ds(d * _LANES, _LANES)
                ws = [rows[buf, r0 + j, s] for j in range(_K)]
                los = [
                    lax.bitcast_convert_type(lax.shift_left(w, 16), jnp.float32)
                    for w in ws
                ]
                his = [
                    lax.bitcast_convert_type(w & _HIMASK, jnp.float32)
                    for w in ws
                ]
                while len(his) > 1:
                    his = [
                        jnp.maximum(his[k], his[k + 1])
                        for k in range(0, len(his), 2)
                    ]
                    los = [
                        jnp.maximum(los[k], los[k + 1])
                        for k in range(0, len(los), 2)
                    ]
                outv[buf, g, s] = los[0]
                outv[buf, g, pl.ds(_FH + d * _LANES, _LANES)] = his[0]
                return 0

            return lax.fori_loop(0, _FH // _LANES, per_d, 0)

        lax.fori_loop(0, _G, per_g, 0)

    def put_out(c, buf):
        base = wid * _RPW + c * _G
        pltpu.async_copy(outv.at[buf], out.at[pl.ds(base, _G)], osems[buf])

    for b in range(_NBUF - 1):
        start_gather(b, b)

    def step(i, _):
        c0 = i * _NBUF
        for b in range(_NBUF):
            c = c0 + b
            nxt = c + _NBUF - 1

            @pl.when(nxt < _NCH)
            def _():
                start_gather(nxt, (b + _NBUF - 1) % _NBUF)

            wait_gather(c, b)
            compute(c, b)
            put_out(c, b)
        return 0

    lax.fori_loop(0, _NCH // _NBUF, step, 0)
    for b in range(_NBUF):
        wait_out(b)


def _rnd(x):
    u = lax.bitcast_convert_type(x, jnp.uint32)
    return (u + jnp.uint32(0x7FFF) + ((u >> 16) & jnp.uint32(1))) >> 16


def _pack(inp_half):
    lo = _rnd(inp_half[:, :_FH])
    hi = _rnd(inp_half[:, _FH:])
    return lax.bitcast_convert_type(lo | (hi << 16), jnp.int32)


_mesh = plsc.VectorSubcoreMesh(core_axis_name="c", subcore_axis_name="s")
_kern_half = pl.kernel(
    _body,
    mesh=_mesh,
    out_type=jax.ShapeDtypeStruct((_H * _M, _F), jnp.float32),
    scratch_types=[
        pltpu.VMEM((_NCH, _IPC), jnp.int32),
        pltpu.VMEM((_NBUF, _IPC, _FH), jnp.int32),
        pltpu.VMEM((_NBUF, _G, _F), jnp.float32),
        pltpu.SemaphoreType.DMA,
        pltpu.SemaphoreType.DMA,
        pltpu.SemaphoreType.DMA,
        pltpu.SemaphoreType.DMA,
        pltpu.SemaphoreType.DMA,
        pltpu.SemaphoreType.DMA,
        pltpu.SemaphoreType.DMA,
        pltpu.SemaphoreType.DMA,
    ],
)


@jax.jit
def kernel(inputs, batch_index):
    inp = inputs.reshape(_B * _N, _F)
    idx = batch_index.reshape(2, _NW, _NCH, _IPC)
    outs = []
    for h in range(2):
        tbl = _pack(inp[h * _H * _N:(h + 1) * _H * _N])
        if outs:
            tbl, _ = lax.optimization_barrier((tbl, outs[0]))
        outs.append(_kern_half(tbl, idx[h]))
    return jnp.concatenate(outs, axis=0).reshape(_B, _M, _F)


# speedup vs baseline: 1.3074x; 1.3074x over previous
import functools

import jax
import jax.numpy as jnp
from jax import lax
from jax.experimental import pallas as pl
from jax.experimental.pallas import tpu as pltpu
from jax.experimental.pallas import tpu_sc as plsc

_B = 16
_M = 1024
_K = 16
_F = 256
_N = 4096

_FH = _F // 2
_NC = 2
_NS = 16
_NW = _NC * _NS
_RPW = (_B * _M) // _NW
_G = 8
_NCH = _RPW // _G
_IPC = _G * _K
_LANES = 16
_NBUF = 4

_HIMASK = -65536


def _body(
    tbl, idx, out, idxv, rows, outv,
    gsem0, gsem1, gsem2, gsem3, osem0, osem1, osem2, osem3,
):
    wid = lax.axis_index("s") * _NC + lax.axis_index("c")
    boff = (wid // (_NW // _B)) * _N

    pltpu.sync_copy(idx.at[wid], idxv)

    def add_off(i, _):
        for j in range(_IPC // _LANES):
            sl = (i, pl.ds(j * _LANES, _LANES))
            idxv[sl] = idxv[sl] + boff
        return 0

    lax.fori_loop(0, _NCH, add_off, 0)

    gsems = (gsem0, gsem1, gsem2, gsem3)
    osems = (osem0, osem1, osem2, osem3)

    def start_gather(c, buf):
        pltpu.async_copy(tbl.at[idxv.at[c]], rows.at[buf], gsems[buf])

    def wait_gather(c, buf):
        pltpu.make_async_copy(tbl.at[idxv.at[c]], rows.at[buf], gsems[buf]).wait()

    def wait_out(buf):
        pltpu.make_async_copy(outv.at[buf], out.at[pl.ds(0, _G)], osems[buf]).wait()

    def compute(c, buf):
        @pl.when(c >= _NBUF)
        def _():
            wait_out(buf)

        def per_g(g, _):
            r0 = g * _K

            def per_d(d, __):
                s = pl.ds(d * _LANES, _LANES)
                ws = [rows[buf, r0 + j, s] for j in range(_K)]
                los = [
                    lax.bitcast_convert_type(lax.shift_left(w, 16), jnp.float32)
                    for w in ws
                ]
                his = [lax.bitcast_convert_type(w, jnp.float32) for w in ws]
                while len(his) > 1:
                    his = [
                        jnp.maximum(his[k], his[k + 1])
                        for k in range(0, len(his), 2)
                    ]
                    los = [
                        jnp.maximum(los[k], los[k + 1])
                        for k in range(0, len(los), 2)
                    ]
                hi = lax.bitcast_convert_type(
                    lax.bitcast_convert_type(his[0], jnp.int32) & _HIMASK,
                    jnp.float32,
                )
                outv[buf, g, s] = los[0]
                outv[buf, g, pl.ds(_FH + d * _LANES, _LANES)] = hi
                return 0

            return lax.fori_loop(0, _FH // _LANES, per_d, 0)

        lax.fori_loop(0, _G, per_g, 0)

    def put_out(c, buf):
        base = wid * _RPW + c * _G
        pltpu.async_copy(outv.at[buf], out.at[pl.ds(base, _G)], osems[buf])

    for b in range(_NBUF - 1):
        start_gather(b, b)

    def step(i, _):
        c0 = i * _NBUF
        for b in range(_NBUF):
            c = c0 + b
            nxt = c + _NBUF - 1

            @pl.when(nxt < _NCH)
            def _():
                start_gather(nxt, (b + _NBUF - 1) % _NBUF)

            wait_gather(c, b)
            compute(c, b)
            put_out(c, b)
        return 0

    lax.fori_loop(0, _NCH // _NBUF, step, 0)
    for b in range(_NBUF):
        wait_out(b)


@jax.jit
def kernel(inputs, batch_index):
    inp = inputs.reshape(_B * _N, _F)

    def _rnd(x):
        u = lax.bitcast_convert_type(x, jnp.uint32)
        return (u + jnp.uint32(0x7FFF) + ((u >> 16) & jnp.uint32(1))) >> 16

    lo = _rnd(inp[:, :_FH])
    hi = _rnd(inp[:, _FH:])
    tbl = lax.bitcast_convert_type(lo | (hi << 16), jnp.int32)

    idx = batch_index.reshape(_NW, _NCH, _IPC)
    mesh = plsc.VectorSubcoreMesh(core_axis_name="c", subcore_axis_name="s")
    kern = pl.kernel(
        _body,
        mesh=mesh,
        out_type=jax.ShapeDtypeStruct((_B * _M, _F), jnp.float32),
        scratch_types=[
            pltpu.VMEM((_NCH, _IPC), jnp.int32),
            pltpu.VMEM((_NBUF, _IPC, _FH), jnp.int32),
            pltpu.VMEM((_NBUF, _G, _F), jnp.float32),
            pltpu.SemaphoreType.DMA,
            pltpu.SemaphoreType.DMA,
            pltpu.SemaphoreType.DMA,
            pltpu.SemaphoreType.DMA,
            pltpu.SemaphoreType.DMA,
            pltpu.SemaphoreType.DMA,
            pltpu.SemaphoreType.DMA,
            pltpu.SemaphoreType.DMA,
        ],
    )
    out = kern(tbl, idx)
    return out.reshape(_B, _M, _F)
